# baseline (speedup 1.0000x reference)
import jax
import jax.numpy as jnp
from jax.experimental import pallas as pl


def _copy_body(i_ref, o_ref):
    o_ref[...] = i_ref[...]


def _blocked_copy(a, grid):
    rb = a.shape[0] // grid
    return pl.pallas_call(
        _copy_body,
        grid=(grid,),
        in_specs=[pl.BlockSpec((rb, a.shape[1]), lambda i: (i, 0))],
        out_specs=pl.BlockSpec((rb, a.shape[1]), lambda i: (i, 0)),
        out_shape=jax.ShapeDtypeStruct(a.shape, a.dtype),
    )(a)


def kernel(x, edge_index, edge_attr):
    del edge_index
    return (_blocked_copy(x, 10), _blocked_copy(edge_attr, 40))
